# Initial kernel scaffold; baseline (speedup 1.0000x reference)
import jax
import jax.numpy as jnp
from jax.experimental import pallas as pl


def kernel(classifications, regressions, anchors, annotations):
    raise NotImplementedError("write your pallas kernel here")


# speedup vs baseline: 5.2525x; 5.2525x over previous
import functools

import jax
import jax.numpy as jnp
from jax.experimental import pallas as pl
from jax.experimental.pallas import tpu as pltpu


def _body(cls_ref, regt_ref, anct_ref, ann_ref, loss_ref, te_ref,
          acc_ref, seg_ref, *, nblk, m, c, a_blk):
    i = pl.program_id(1)

    @pl.when(i == 0)
    def _init():
        acc_ref[0] = 0.0
        acc_ref[1] = 0.0
        acc_ref[2] = 0.0
        seg_ref[...] = jnp.zeros_like(seg_ref)

    anct = anct_ref[...]
    ann = ann_ref[0]
    p = jnp.clip(cls_ref[0], 1e-4, 1.0 - 1e-4)
    regt = regt_ref[0]

    ay1 = anct[0:1, :]
    ax1 = anct[1:2, :]
    ay2 = anct[2:3, :]
    ax2 = anct[3:4, :]

    bx1 = ann[:, 0:1]
    by1 = ann[:, 1:2]
    bx2 = ann[:, 2:3]
    by2 = ann[:, 3:4]
    bcls = ann[:, 4:5]
    valid = bcls != -1.0

    area_b = (bx2 - bx1) * (by2 - by1)
    iw = jnp.maximum(jnp.minimum(ax2, bx2) - jnp.maximum(ax1, bx1), 0.0)
    ih = jnp.maximum(jnp.minimum(ay2, by2) - jnp.maximum(ay1, by1), 0.0)
    inter = iw * ih
    area_a = (ay2 - ay1) * (ax2 - ax1)
    ua = jnp.maximum(area_a + area_b - inter, 1e-8)
    iou = jnp.where(valid, inter / ua, -1.0)

    iou_max = jnp.max(iou, axis=0)
    iota_m = jax.lax.broadcasted_iota(jnp.int32, (m, a_blk), 0)
    idx = jnp.min(jnp.where(iou == iou_max[None, :], iota_m, m), axis=0)
    oh = jnp.where(iota_m == idx[None, :], 1.0, 0.0)

    pos = iou_max >= 0.5
    use = pos | (iou_max < 0.4)
    posf = pos.astype(jnp.float32)
    npos_blk = jnp.sum(posf)

    assigned = jax.lax.dot_general(
        ann.T, oh, (((1,), (0,)), ((), ())),
        preferred_element_type=jnp.float32)

    lneg = 0.75 * p * p * (-jnp.log(1.0 - p))
    rsum = jnp.sum(lneg, axis=1)

    cls_idx = assigned[4, :].astype(jnp.int32)
    iota_c = jax.lax.broadcasted_iota(jnp.int32, (a_blk, c), 1)
    p_sel = jnp.sum(jnp.where(iota_c == cls_idx[:, None], p, 0.0), axis=1)
    p_sel = jnp.clip(p_sel, 1e-4, 1.0 - 1e-4)
    lneg_sel = 0.75 * p_sel * p_sel * (-jnp.log(1.0 - p_sel))
    lpos_sel = 0.25 * (1.0 - p_sel) ** 2 * (-jnp.log(p_sel))

    cls_blk = (jnp.sum(jnp.where(use, rsum, 0.0))
               + jnp.sum(jnp.where(pos, lpos_sel - lneg_sel, 0.0)))

    aw = ax2[0] - ax1[0]
    ah = ay2[0] - ay1[0]
    acx = ax1[0] + 0.5 * aw
    acy = ay1[0] + 0.5 * ah
    gx1 = assigned[0, :]
    gy1 = assigned[1, :]
    gw = assigned[2, :] - gx1
    gh = assigned[3, :] - gy1
    gcx = gx1 + 0.5 * gw
    gcy = gy1 + 0.5 * gh
    gw = jnp.maximum(gw, 1.0)
    gh = jnp.maximum(gh, 1.0)
    tdy = (gcy - acy) / ah
    tdx = (gcx - acx) / aw
    tdh = jnp.log(gh / ah)
    tdw = jnp.log(gw / aw)

    def sl1(t, r):
        d = jnp.abs(t - r)
        return jnp.where(d <= 1.0 / 9.0, 0.5 * 9.0 * d * d, d - 0.5 / 9.0)

    rl = (sl1(tdy, regt[0, :]) + sl1(tdx, regt[1, :])
          + sl1(tdh, regt[2, :]) + sl1(tdw, regt[3, :]))
    reg_blk = jnp.sum(jnp.where(pos, rl, 0.0))

    x = regt[4:7, :]
    sq = jnp.sum(x * x, axis=0, keepdims=True)
    ones = jnp.ones((1, a_blk), jnp.float32)
    feats = jnp.concatenate([ones, x, sq], axis=0)
    oh_pos = oh * posf[None, :]
    seg_blk = jax.lax.dot_general(
        oh_pos, feats, (((1,), (1,)), ((), ())),
        preferred_element_type=jnp.float32)
    seg_ref[...] += seg_blk

    acc_ref[0] += cls_blk
    acc_ref[1] += npos_blk
    acc_ref[2] += reg_blk

    @pl.when(i == nblk - 1)
    def _fin():
        npos = acc_ref[1]
        cls_loss = acc_ref[0] / jnp.maximum(npos, 1.0)
        reg_loss = jnp.where(npos > 0.0, acc_ref[2] / (npos * 4.0), 0.0)
        seg = seg_ref[...]
        cnt = seg[:, 0:1]
        cnt_ok = cnt > 0.0
        cnt_safe = jnp.where(cnt_ok, cnt, 1.0)
        s = seg[:, 1:4]
        sqs = seg[:, 4:5]
        te = jnp.where(cnt_ok, s / cnt_safe, 0.0)
        s2 = jnp.sum(s * s, axis=1, keepdims=True)
        contrib = jnp.where(cnt_ok, (sqs - s2 / cnt_safe) / (cnt_safe * 3.0),
                            0.0)
        emb_loss = jnp.sum(contrib) / float(m)
        li = jax.lax.broadcasted_iota(jnp.int32, (1, 1, 3), 2)
        loss_ref[...] = jnp.where(
            li == 0, cls_loss, jnp.where(li == 1, reg_loss, emb_loss))
        te_ref[0] = te


@jax.jit
def kernel(classifications, regressions, anchors, annotations):
    b, a, c = classifications.shape
    m = annotations.shape[1]
    a_blk = 8192
    if a % a_blk:
        a_blk = a
    nblk = a // a_blk

    anct = anchors[0].T
    regt = jnp.transpose(regressions, (0, 2, 1))

    losses_img, te = pl.pallas_call(
        functools.partial(_body, nblk=nblk, m=m, c=c, a_blk=a_blk),
        grid=(b, nblk),
        in_specs=[
            pl.BlockSpec((1, a_blk, c), lambda bi, i: (bi, i, 0)),
            pl.BlockSpec((1, 7, a_blk), lambda bi, i: (bi, 0, i)),
            pl.BlockSpec((4, a_blk), lambda bi, i: (0, i)),
            pl.BlockSpec((1, m, 5), lambda bi, i: (bi, 0, 0)),
        ],
        out_specs=[
            pl.BlockSpec((1, 1, 3), lambda bi, i: (bi, 0, 0)),
            pl.BlockSpec((1, m, 3), lambda bi, i: (bi, 0, 0)),
        ],
        out_shape=[
            jax.ShapeDtypeStruct((b, 1, 3), jnp.float32),
            jax.ShapeDtypeStruct((b, m, 3), jnp.float32),
        ],
        scratch_shapes=[
            pltpu.SMEM((4,), jnp.float32),
            pltpu.VMEM((m, 5), jnp.float32),
        ],
    )(classifications, regt, anct, annotations)

    losses_img = losses_img[:, 0, :]
    losses = jnp.stack([
        jnp.mean(losses_img[:, 0]),
        jnp.mean(losses_img[:, 1]) * 50.0,
        jnp.mean(losses_img[:, 2]),
    ])
    return losses, te


# speedup vs baseline: 6.0990x; 1.1612x over previous
import functools

import jax
import jax.numpy as jnp
from jax.experimental import pallas as pl
from jax.experimental.pallas import tpu as pltpu


def _match_body(anct_ref, ann_ref, regt_ref, use_ref, clsp_ref, stat_ref,
                te_ref, acc_ref, seg_ref, *, nblk, m, a_blk):
    i = pl.program_id(1)

    @pl.when(i == 0)
    def _init():
        acc_ref[0] = 0.0
        acc_ref[1] = 0.0
        seg_ref[...] = jnp.zeros_like(seg_ref)

    anct = anct_ref[...]
    ann = ann_ref[0]
    regt = regt_ref[0]

    ay1 = anct[0:1, :]
    ax1 = anct[1:2, :]
    ay2 = anct[2:3, :]
    ax2 = anct[3:4, :]

    bx1 = ann[:, 0:1]
    by1 = ann[:, 1:2]
    bx2 = ann[:, 2:3]
    by2 = ann[:, 3:4]
    valid = ann[:, 4:5] != -1.0

    area_b = (bx2 - bx1) * (by2 - by1)
    iw = jnp.maximum(jnp.minimum(ax2, bx2) - jnp.maximum(ax1, bx1), 0.0)
    ih = jnp.maximum(jnp.minimum(ay2, by2) - jnp.maximum(ay1, by1), 0.0)
    inter = iw * ih
    area_a = (ay2 - ay1) * (ax2 - ax1)
    ua = jnp.maximum(area_a + area_b - inter, 1e-8)
    iou = jnp.where(valid, inter / ua, -1.0)

    iou_max = jnp.max(iou, axis=0, keepdims=True)
    iota_m = jax.lax.broadcasted_iota(jnp.int32, (m, a_blk), 0)
    idx = jnp.min(jnp.where(iou == iou_max, iota_m, m), axis=0,
                  keepdims=True)
    oh = jnp.where(iota_m == idx, 1.0, 0.0)

    pos = iou_max >= 0.5
    posf = pos.astype(jnp.float32)
    usef = jnp.where(pos | (iou_max < 0.4), 1.0, 0.0)

    assigned = jax.lax.dot_general(
        ann.T, oh, (((1,), (0,)), ((), ())),
        preferred_element_type=jnp.float32)

    use_ref[0] = usef
    clsp_ref[0] = jnp.where(pos, assigned[4:5, :], -1.0)

    aw = ax2 - ax1
    ah = ay2 - ay1
    acx = ax1 + 0.5 * aw
    acy = ay1 + 0.5 * ah
    gx1 = assigned[0:1, :]
    gy1 = assigned[1:2, :]
    gw = assigned[2:3, :] - gx1
    gh = assigned[3:4, :] - gy1
    gcx = gx1 + 0.5 * gw
    gcy = gy1 + 0.5 * gh
    gw = jnp.maximum(gw, 1.0)
    gh = jnp.maximum(gh, 1.0)
    tdy = (gcy - acy) / ah
    tdx = (gcx - acx) / aw
    tdh = jnp.log(gh / ah)
    tdw = jnp.log(gw / aw)

    def sl1(t_, r_):
        d = jnp.abs(t_ - r_)
        return jnp.where(d <= 1.0 / 9.0, 0.5 * 9.0 * d * d, d - 0.5 / 9.0)

    rl = (sl1(tdy, regt[0:1, :]) + sl1(tdx, regt[1:2, :])
          + sl1(tdh, regt[2:3, :]) + sl1(tdw, regt[3:4, :]))
    reg_blk = jnp.sum(rl * posf)

    x = regt[4:7, :]
    sq = jnp.sum(x * x, axis=0, keepdims=True)
    feats = jnp.concatenate([posf, x, sq], axis=0)
    oh_pos = oh * posf
    seg_blk = jax.lax.dot_general(
        oh_pos, feats, (((1,), (1,)), ((), ())),
        preferred_element_type=jnp.float32)
    seg_ref[...] += seg_blk

    acc_ref[0] += jnp.sum(posf)
    acc_ref[1] += reg_blk

    @pl.when(i == nblk - 1)
    def _fin():
        npos = acc_ref[0]
        reg_loss = jnp.where(npos > 0.0, acc_ref[1] / (npos * 4.0), 0.0)
        seg = seg_ref[...]
        cnt = seg[:, 0:1]
        cnt_ok = cnt > 0.0
        cnt_safe = jnp.where(cnt_ok, cnt, 1.0)
        s = seg[:, 1:4]
        sqs = seg[:, 4:5]
        te = jnp.where(cnt_ok, s / cnt_safe, 0.0)
        s2m = jnp.sum(s * s, axis=1, keepdims=True)
        contrib = jnp.where(cnt_ok, (sqs - s2m / cnt_safe) / (cnt_safe * 3.0),
                            0.0)
        emb_loss = jnp.sum(contrib) / float(m)
        li = jax.lax.broadcasted_iota(jnp.int32, (1, 1, 8), 2)
        stat_ref[...] = jnp.where(
            li == 0, npos, jnp.where(li == 1, reg_loss, emb_loss))
        te_ref[0] = te


def _focal_body(cls_ref, use_ref, clsp_ref, stat_ref, loss_ref, acc_ref,
                *, nblk, c, rows):
    i = pl.program_id(1)

    @pl.when(i == 0)
    def _init():
        acc_ref[0] = 0.0

    lanes = 8 * c
    p = jnp.clip(cls_ref[0], 1e-4, 1.0 - 1e-4)

    i8 = jax.lax.broadcasted_iota(jnp.int32, (8, lanes), 0)
    il = jax.lax.broadcasted_iota(jnp.int32, (8, lanes), 1)
    expand = jnp.where(il // c == i8, 1.0, 0.0)
    zeros8 = jnp.zeros((8, lanes), jnp.float32)
    expand2 = jnp.concatenate([
        jnp.concatenate([expand, zeros8], axis=1),
        jnp.concatenate([zeros8, expand], axis=1)], axis=0)
    um = jnp.concatenate([use_ref[0], clsp_ref[0]], axis=1)
    both = jax.lax.dot_general(
        um, expand2, (((1,), (0,)), ((), ())),
        preferred_element_type=jnp.float32)
    use_d = both[:, :lanes]
    cls_d = both[:, lanes:]

    modc = (il[0:1, :] - (il[0:1, :] // c) * c).astype(jnp.float32)
    eq = cls_d == modc
    oh1 = jnp.where(eq, 1.0, 0.0)
    oh3 = jnp.where(eq, 1.0 / 3.0, 0.0)

    t = 1.0 - p
    s1 = (p * p) * jnp.log(t) * (use_d - oh1)
    s2 = (t * t) * jnp.log(p) * oh3
    acc_ref[0] += jnp.sum(s1 + s2)

    @pl.when(i == nblk - 1)
    def _fin():
        npos = stat_ref[0, 0, 0]
        cls_loss = (-0.75 * acc_ref[0]) / jnp.maximum(npos, 1.0)
        li = jax.lax.broadcasted_iota(jnp.int32, (1, 1, 3), 2)
        loss_ref[...] = jnp.where(
            li == 0, cls_loss,
            jnp.where(li == 1, stat_ref[0, 0, 1], stat_ref[0, 0, 2]))


@jax.jit
def kernel(classifications, regressions, anchors, annotations):
    b, a, c = classifications.shape
    m = annotations.shape[1]
    a_blk = 8192
    if a % a_blk:
        a_blk = a
    nblk = a // a_blk

    anct = anchors[0].T
    regt = jnp.transpose(regressions, (0, 2, 1))

    usef, clsp, stats, te = pl.pallas_call(
        functools.partial(_match_body, nblk=nblk, m=m, a_blk=a_blk),
        grid=(b, nblk),
        in_specs=[
            pl.BlockSpec((4, a_blk), lambda bi, i: (0, i)),
            pl.BlockSpec((1, m, 5), lambda bi, i: (bi, 0, 0)),
            pl.BlockSpec((1, 7, a_blk), lambda bi, i: (bi, 0, i)),
        ],
        out_specs=[
            pl.BlockSpec((1, 1, a_blk), lambda bi, i: (bi, 0, i)),
            pl.BlockSpec((1, 1, a_blk), lambda bi, i: (bi, 0, i)),
            pl.BlockSpec((1, 1, 8), lambda bi, i: (bi, 0, 0)),
            pl.BlockSpec((1, m, 3), lambda bi, i: (bi, 0, 0)),
        ],
        out_shape=[
            jax.ShapeDtypeStruct((b, 1, a), jnp.float32),
            jax.ShapeDtypeStruct((b, 1, a), jnp.float32),
            jax.ShapeDtypeStruct((b, 1, 8), jnp.float32),
            jax.ShapeDtypeStruct((b, m, 3), jnp.float32),
        ],
        scratch_shapes=[
            pltpu.SMEM((2,), jnp.float32),
            pltpu.VMEM((m, 5), jnp.float32),
        ],
    )(anct, annotations, regt)

    rows = a_blk // 8
    losses_img = pl.pallas_call(
        functools.partial(_focal_body, nblk=nblk, c=c, rows=rows),
        grid=(b, nblk),
        in_specs=[
            pl.BlockSpec((1, rows, 8 * c), lambda bi, i: (bi, i, 0)),
            pl.BlockSpec((1, rows, 8), lambda bi, i: (bi, i, 0)),
            pl.BlockSpec((1, rows, 8), lambda bi, i: (bi, i, 0)),
            pl.BlockSpec((1, 1, 8), lambda bi, i: (bi, 0, 0)),
        ],
        out_specs=pl.BlockSpec((1, 1, 3), lambda bi, i: (bi, 0, 0)),
        out_shape=jax.ShapeDtypeStruct((b, 1, 3), jnp.float32),
        scratch_shapes=[pltpu.SMEM((1,), jnp.float32)],
    )(classifications.reshape(b, a // 8, 8 * c),
      usef.reshape(b, a // 8, 8),
      clsp.reshape(b, a // 8, 8),
      stats)

    losses_img = losses_img[:, 0, :]
    losses = jnp.stack([
        jnp.mean(losses_img[:, 0]),
        jnp.mean(losses_img[:, 1]) * 50.0,
        jnp.mean(losses_img[:, 2]),
    ])
    return losses, te
